# baseline (speedup 1.0000x reference)
import jax
import jax.numpy as jnp
from jax.experimental import pallas as pl
from jax.experimental.pallas import tpu as pltpu

B, K, IN, H, OUT = 8, 256, 128, 64, 128
NG = 2
GRP = B // NG
ALPHA = 0.2
NEG_BIG = -9000000000000000.0
LOG2E = 1.4426950408889634


def _gat_block(h, mask, W_ref, a_ref, ones, nh):
    Wh = jax.lax.dot_general(h, W_ref[...], (((1,), (0,)), ((), ())),
                             preferred_element_type=jnp.float32)
    a_s = a_ref[:, :nh] * LOG2E
    a_d = a_ref[:, nh:] * LOG2E
    s = jax.lax.dot_general(Wh, a_s, (((1,), (1,)), ((), ())),
                            preferred_element_type=jnp.float32)
    d = jax.lax.dot_general(a_d, Wh, (((1,), (1,)), ((), ())),
                            preferred_element_type=jnp.float32)
    e = s + d
    e = jnp.maximum(e, ALPHA * e)
    att = jnp.where(mask, e, NEG_BIG)
    m = jnp.max(att, axis=1, keepdims=True)
    p = jnp.exp2(att - m)
    rs = jax.lax.dot_general(p, ones, (((1,), (1,)), ((), ())),
                             preferred_element_type=jnp.float32)
    hp = jax.lax.dot_general(p, Wh, (((1,), (0,)), ((), ())),
                             preferred_element_type=jnp.float32)
    hp = hp * (1.0 / rs)
    return jnp.where(hp > 0, hp, jnp.exp(jnp.minimum(hp, 0.0)) - 1.0)


def _gat2_kernel(x_hbm, adj_hbm, W1_ref, a1_ref, W2_ref, a2_ref, out_hbm,
                 x_vm, adj_vm, out_vm, sem_x, sem_a, sem_o):
    ones = jnp.ones((1, K), dtype=jnp.float32)
    for g in range(NG):
        sl = pl.ds(g * GRP, GRP)
        pltpu.make_async_copy(x_hbm.at[sl], x_vm.at[sl], sem_x.at[g]).start()
        pltpu.make_async_copy(adj_hbm.at[sl], adj_vm.at[sl],
                              sem_a.at[g]).start()
    for g in range(NG):
        sl = pl.ds(g * GRP, GRP)
        pltpu.make_async_copy(x_hbm.at[sl], x_vm.at[sl], sem_x.at[g]).wait()
        pltpu.make_async_copy(adj_hbm.at[sl], adj_vm.at[sl],
                              sem_a.at[g]).wait()
        for i in range(g * GRP, (g + 1) * GRP):
            x = x_vm[i]
            mask = adj_vm[i] > 0
            h1 = _gat_block(x, mask, W1_ref, a1_ref, ones, H)
            out_vm[i] = _gat_block(h1, mask, W2_ref, a2_ref, ones, OUT)
        pltpu.make_async_copy(out_vm.at[sl], out_hbm.at[sl],
                              sem_o.at[g]).start()
    for g in range(NG):
        sl = pl.ds(g * GRP, GRP)
        pltpu.make_async_copy(out_vm.at[sl], out_hbm.at[sl],
                              sem_o.at[g]).wait()


def kernel(x, adj, W1, a1, W2, a2):
    out = pl.pallas_call(
        _gat2_kernel,
        in_specs=[
            pl.BlockSpec(memory_space=pltpu.MemorySpace.HBM),
            pl.BlockSpec(memory_space=pltpu.MemorySpace.HBM),
            pl.BlockSpec((IN, H), lambda: (0, 0)),
            pl.BlockSpec((1, 2 * H), lambda: (0, 0)),
            pl.BlockSpec((H, OUT), lambda: (0, 0)),
            pl.BlockSpec((1, 2 * OUT), lambda: (0, 0)),
        ],
        out_specs=pl.BlockSpec(memory_space=pltpu.MemorySpace.HBM),
        out_shape=jax.ShapeDtypeStruct((B, K, OUT), jnp.float32),
        scratch_shapes=[
            pltpu.VMEM((B, K, IN), jnp.float32),
            pltpu.VMEM((B, K, K), jnp.float32),
            pltpu.VMEM((B, K, OUT), jnp.float32),
            pltpu.SemaphoreType.DMA((NG,)),
            pltpu.SemaphoreType.DMA((NG,)),
            pltpu.SemaphoreType.DMA((NG,)),
        ],
    )(x, adj, W1, a1.reshape(1, 2 * H), W2, a2.reshape(1, 2 * OUT))
    return out


# speedup vs baseline: 1.0778x; 1.0778x over previous
import jax
import jax.numpy as jnp
from jax.experimental import pallas as pl

B, K, IN, H, OUT = 8, 256, 128, 64, 128
ALPHA = 0.2
NEG_BIG = -9000000000000000.0
LOG2E = 1.4426950408889634


def _gat_block(h, mask, W_ref, a_ref, ones, nh):
    Wh = jax.lax.dot_general(h, W_ref[...], (((1,), (0,)), ((), ())),
                             preferred_element_type=jnp.float32)
    a_s = a_ref[:, :nh] * LOG2E
    a_d = a_ref[:, nh:] * LOG2E
    s = jax.lax.dot_general(Wh, a_s, (((1,), (1,)), ((), ())),
                            preferred_element_type=jnp.float32)
    d = jax.lax.dot_general(a_d, Wh, (((1,), (1,)), ((), ())),
                            preferred_element_type=jnp.float32)
    e = s + d
    e = jnp.maximum(e, ALPHA * e)
    att = jnp.where(mask, e, NEG_BIG)
    m = jnp.max(att, axis=1, keepdims=True)
    p = jnp.exp2(att - m)
    rs = jax.lax.dot_general(p, ones, (((1,), (1,)), ((), ())),
                             preferred_element_type=jnp.float32)
    hp = jax.lax.dot_general(p, Wh, (((1,), (0,)), ((), ())),
                             preferred_element_type=jnp.float32)
    hp = hp * (1.0 / rs)
    return jnp.where(hp > 0, hp, jnp.exp(jnp.minimum(hp, 0.0)) - 1.0)


def _gat2_kernel(x_ref, adj_ref, W1_ref, a1_ref, W2_ref, a2_ref, out_ref):
    ones = jnp.ones((1, K), dtype=jnp.float32)
    for i in range(B):
        x = x_ref[i]
        mask = adj_ref[i] > 0
        h1 = _gat_block(x, mask, W1_ref, a1_ref, ones, H)
        out_ref[i] = _gat_block(h1, mask, W2_ref, a2_ref, ones, OUT)


def kernel(x, adj, W1, a1, W2, a2):
    out = pl.pallas_call(
        _gat2_kernel,
        in_specs=[
            pl.BlockSpec((B, K, IN), lambda: (0, 0, 0)),
            pl.BlockSpec((B, K, K), lambda: (0, 0, 0)),
            pl.BlockSpec((IN, H), lambda: (0, 0)),
            pl.BlockSpec((1, 2 * H), lambda: (0, 0)),
            pl.BlockSpec((H, OUT), lambda: (0, 0)),
            pl.BlockSpec((1, 2 * OUT), lambda: (0, 0)),
        ],
        out_specs=pl.BlockSpec((B, K, OUT), lambda: (0, 0, 0)),
        out_shape=jax.ShapeDtypeStruct((B, K, OUT), jnp.float32),
    )(x, adj, W1, a1.reshape(1, 2 * H), W2, a2.reshape(1, 2 * OUT))
    return out


# speedup vs baseline: 1.0805x; 1.0025x over previous
import jax
import jax.numpy as jnp
from jax.experimental import pallas as pl

B, K, IN, H, OUT = 8, 256, 128, 64, 128
ALPHA = 0.2
NEG_BIG = -9000000000000000.0
LOG2E = 1.4426950408889634


def _gat_block(h, adj_ref, i, W_ref, a_ref, ones, nh):
    Wh = jax.lax.dot_general(h, W_ref[...], (((1,), (0,)), ((), ())),
                             preferred_element_type=jnp.float32)
    a_s = a_ref[...][None, :nh] * LOG2E
    a_d = a_ref[...][None, nh:] * LOG2E
    s = jax.lax.dot_general(Wh, a_s, (((1,), (1,)), ((), ())),
                            preferred_element_type=jnp.float32)
    d = jax.lax.dot_general(a_d, Wh, (((1,), (1,)), ((), ())),
                            preferred_element_type=jnp.float32)
    e = s + d
    e = jnp.maximum(e, ALPHA * e)
    att = jnp.where(adj_ref[i] > 0, e, NEG_BIG)
    m = jnp.max(att, axis=1, keepdims=True)
    p = jnp.exp2(att - m)
    rs = jax.lax.dot_general(p, ones, (((1,), (1,)), ((), ())),
                             preferred_element_type=jnp.float32)
    hp = jax.lax.dot_general(p, Wh, (((1,), (0,)), ((), ())),
                             preferred_element_type=jnp.float32)
    hp = hp * (1.0 / rs)
    return jnp.where(hp > 0, hp, jnp.exp(jnp.minimum(hp, 0.0)) - 1.0)


def _gat2_kernel(x_ref, adj_ref, W1_ref, a1_ref, W2_ref, a2_ref, out_ref):
    ones = jnp.ones((1, K), dtype=jnp.float32)
    for i in range(B):
        x = x_ref[i]
        h1 = _gat_block(x, adj_ref, i, W1_ref, a1_ref, ones, H)
        out_ref[i] = _gat_block(h1, adj_ref, i, W2_ref, a2_ref, ones, OUT)


def kernel(x, adj, W1, a1, W2, a2):
    out = pl.pallas_call(
        _gat2_kernel,
        in_specs=[
            pl.BlockSpec((B, K, IN), lambda: (0, 0, 0)),
            pl.BlockSpec((B, K, K), lambda: (0, 0, 0)),
            pl.BlockSpec((IN, H), lambda: (0, 0)),
            pl.BlockSpec((2 * H,), lambda: (0,)),
            pl.BlockSpec((H, OUT), lambda: (0, 0)),
            pl.BlockSpec((2 * OUT,), lambda: (0,)),
        ],
        out_specs=pl.BlockSpec((B, K, OUT), lambda: (0, 0, 0)),
        out_shape=jax.ShapeDtypeStruct((B, K, OUT), jnp.float32),
    )(x, adj, W1, a1, W2, a2)
    return out


# speedup vs baseline: 1.4215x; 1.3156x over previous
import jax
import jax.numpy as jnp
from jax.experimental import pallas as pl

B, K, IN, H, OUT = 8, 256, 128, 64, 128
ALPHA = 0.2
NEG_BIG = -9000000000000000.0
LOG2E = 1.4426950408889634


def _gat_layer_all(h_all, adj_ref, W_ref, a_ref, ones, nh):
    Wh_all = jax.lax.dot_general(h_all, W_ref[...], (((1,), (0,)), ((), ())),
                                 preferred_element_type=jnp.float32)
    a_s = a_ref[...][None, :nh] * LOG2E
    a_d = a_ref[...][None, nh:] * LOG2E
    s_all = jax.lax.dot_general(Wh_all, a_s, (((1,), (1,)), ((), ())),
                                preferred_element_type=jnp.float32)
    d_all = jax.lax.dot_general(a_d, Wh_all, (((1,), (1,)), ((), ())),
                                preferred_element_type=jnp.float32)
    outs = []
    for i in range(B):
        lo = i * K
        e = s_all[lo:lo + K] + d_all[:, lo:lo + K]
        e = jnp.maximum(e, ALPHA * e)
        att = jnp.where(adj_ref[i] > 0, e, NEG_BIG)
        m = jnp.max(att, axis=1, keepdims=True)
        p = jnp.exp2(att - m)
        rs = jax.lax.dot_general(p, ones, (((1,), (1,)), ((), ())),
                                 preferred_element_type=jnp.float32)
        hp = jax.lax.dot_general(p, Wh_all[lo:lo + K],
                                 (((1,), (0,)), ((), ())),
                                 preferred_element_type=jnp.float32)
        hp = hp * (1.0 / rs)
        outs.append(jnp.where(hp > 0, hp,
                              jnp.exp(jnp.minimum(hp, 0.0)) - 1.0))
    return jnp.concatenate(outs, axis=0)


def _gat2_kernel(x_ref, adj_ref, W1_ref, a1_ref, W2_ref, a2_ref, out_ref):
    ones = jnp.ones((1, K), dtype=jnp.float32)
    x_all = x_ref[...].reshape(B * K, IN)
    h1_all = _gat_layer_all(x_all, adj_ref, W1_ref, a1_ref, ones, H)
    out_all = _gat_layer_all(h1_all, adj_ref, W2_ref, a2_ref, ones, OUT)
    out_ref[...] = out_all.reshape(B, K, OUT)


def kernel(x, adj, W1, a1, W2, a2):
    out = pl.pallas_call(
        _gat2_kernel,
        in_specs=[
            pl.BlockSpec((B, K, IN), lambda: (0, 0, 0)),
            pl.BlockSpec((B, K, K), lambda: (0, 0, 0)),
            pl.BlockSpec((IN, H), lambda: (0, 0)),
            pl.BlockSpec((2 * H,), lambda: (0,)),
            pl.BlockSpec((H, OUT), lambda: (0, 0)),
            pl.BlockSpec((2 * OUT,), lambda: (0,)),
        ],
        out_specs=pl.BlockSpec((B, K, OUT), lambda: (0, 0, 0)),
        out_shape=jax.ShapeDtypeStruct((B, K, OUT), jnp.float32),
    )(x, adj, W1, a1, W2, a2)
    return out


# speedup vs baseline: 1.4374x; 1.0112x over previous
import jax
import jax.numpy as jnp
from jax.experimental import pallas as pl

B, K, IN, H, OUT = 8, 256, 128, 64, 128
ALPHA = 0.2
NEG_BIG = -9000000000000000.0
LOG2E = 1.4426950408889634


def _gat_layer_all(h_all, adj_ref, W_ref, a_ref, ones, nh):
    Wh_all = jax.lax.dot_general(h_all, W_ref[...], (((1,), (0,)), ((), ())),
                                 preferred_element_type=jnp.float32)
    a_s = a_ref[...][None, :nh] * LOG2E
    a_d = a_ref[...][None, nh:] * LOG2E
    s_all = jax.lax.dot_general(Wh_all, a_s, (((1,), (1,)), ((), ())),
                                preferred_element_type=jnp.float32)
    d_all = jax.lax.dot_general(a_d, Wh_all, (((1,), (1,)), ((), ())),
                                preferred_element_type=jnp.float32)
    Whx_all = jnp.concatenate(
        [Wh_all, jnp.ones((B * K, 1), jnp.float32)], axis=1)
    outs = []
    for i in range(B):
        lo = i * K
        e = s_all[lo:lo + K] + d_all[:, lo:lo + K]
        e = jnp.maximum(e, ALPHA * e)
        att = jnp.where(adj_ref[i] > 0, e, NEG_BIG)
        m = jnp.max(att, axis=1, keepdims=True)
        p = jnp.exp2(att - m)
        hpx = jax.lax.dot_general(p, Whx_all[lo:lo + K],
                                  (((1,), (0,)), ((), ())),
                                  preferred_element_type=jnp.float32)
        hp = hpx[:, :nh] * (1.0 / hpx[:, nh:nh + 1])
        outs.append(jnp.where(hp > 0, hp,
                              jnp.exp(jnp.minimum(hp, 0.0)) - 1.0))
    return jnp.concatenate(outs, axis=0)


def _gat2_kernel(x_ref, adj_ref, W1_ref, a1_ref, W2_ref, a2_ref, out_ref):
    ones = jnp.ones((1, K), dtype=jnp.float32)
    x_all = x_ref[...].reshape(B * K, IN)
    h1_all = _gat_layer_all(x_all, adj_ref, W1_ref, a1_ref, ones, H)
    out_all = _gat_layer_all(h1_all, adj_ref, W2_ref, a2_ref, ones, OUT)
    out_ref[...] = out_all.reshape(B, K, OUT)


def kernel(x, adj, W1, a1, W2, a2):
    out = pl.pallas_call(
        _gat2_kernel,
        in_specs=[
            pl.BlockSpec((B, K, IN), lambda: (0, 0, 0)),
            pl.BlockSpec((B, K, K), lambda: (0, 0, 0)),
            pl.BlockSpec((IN, H), lambda: (0, 0)),
            pl.BlockSpec((2 * H,), lambda: (0,)),
            pl.BlockSpec((H, OUT), lambda: (0, 0)),
            pl.BlockSpec((2 * OUT,), lambda: (0,)),
        ],
        out_specs=pl.BlockSpec((B, K, OUT), lambda: (0, 0, 0)),
        out_shape=jax.ShapeDtypeStruct((B, K, OUT), jnp.float32),
    )(x, adj, W1, a1, W2, a2)
    return out
